# Initial kernel scaffold; baseline (speedup 1.0000x reference)
import jax
import jax.numpy as jnp
from jax.experimental import pallas as pl


def kernel(feats, edge_index, W1, b1, W2, b2):
    raise NotImplementedError("write your pallas kernel here")


# speedup vs baseline: 2.9664x; 2.9664x over previous
import functools

import jax
import jax.numpy as jnp
from jax import lax
from jax.experimental import pallas as pl
from jax.experimental.pallas import tpu as pltpu
from jax.experimental.pallas import tpu_sc as plsc

N_NODES = 10000
D = 128
N_EDGES = 320000

NC = 2
NS = 16
NW = NC * NS

NPAD = 10240
EPW = 10240
EPAD = NW * EPW
CHUNK = 128
NCHUNK = EPW // CHUNK
ROWS_PER_TILE = NPAD // NS

DEG_CHUNKS = 2 * EPAD // (NW * CHUNK)

_MESH = plsc.VectorSubcoreMesh(core_axis_name="c", subcore_axis_name="s")


def _zero_vmem_2d(ref, n_rows):
    def body(i, carry):
        r = i // 8
        c = (i % 8) * 16
        ref[r, pl.ds(c, 16)] = jnp.zeros((16,), jnp.float32)
        return carry
    lax.fori_loop(0, n_rows * 8, body, 0)


def _zero_vmem_1d(ref, n):
    def body(i, carry):
        ref[pl.ds(i * 16, 16)] = jnp.zeros((16,), jnp.float32)
        return carry
    lax.fori_loop(0, n // 16, body, 0)


@functools.partial(
    pl.kernel,
    out_type=jax.ShapeDtypeStruct((NC, 2 * NPAD), jnp.float32),
    mesh=_MESH,
    scratch_types=[
        pltpu.VMEM((CHUNK,), jnp.int32),
        pltpu.VMEM((CHUNK,), jnp.float32),
        pltpu.VMEM_SHARED((2 * NPAD,), jnp.float32),
    ],
)
def _sc_degrees(deg_idx_hbm, out_hbm, idx_v, ones_v, deg_sh):
    cid = lax.axis_index("c")
    sid = lax.axis_index("s")
    wid = sid * NC + cid

    def ones_body(i, carry):
        ones_v[pl.ds(i * 16, 16)] = jnp.ones((16,), jnp.float32)
        return carry
    lax.fori_loop(0, CHUNK // 16, ones_body, 0)

    per_tile = (2 * NPAD) // NS
    zbuf = ones_v
    _zero_vmem_1d(zbuf, CHUNK)
    def zslice(r, carry):
        pltpu.sync_copy(zbuf, deg_sh.at[pl.ds(sid * per_tile + r * CHUNK, CHUNK)])
        return carry
    lax.fori_loop(0, per_tile // CHUNK, zslice, 0)
    def ones_body2(i, carry):
        ones_v[pl.ds(i * 16, 16)] = jnp.ones((16,), jnp.float32)
        return carry
    lax.fori_loop(0, CHUNK // 16, ones_body2, 0)

    plsc.subcore_barrier()

    def body(j, carry):
        pltpu.sync_copy(deg_idx_hbm.at[wid * DEG_CHUNKS + j], idx_v)
        pltpu.sync_copy(ones_v, deg_sh.at[idx_v], add=True)
        return carry
    lax.fori_loop(0, DEG_CHUNKS, body, 0)

    plsc.subcore_barrier()
    pltpu.sync_copy(
        deg_sh.at[pl.ds(sid * per_tile, per_tile)],
        out_hbm.at[cid, pl.ds(sid * per_tile, per_tile)],
    )


@functools.partial(
    pl.kernel,
    out_type=jax.ShapeDtypeStruct((NC * NPAD, D), jnp.float32),
    mesh=_MESH,
    scratch_types=[
        pltpu.VMEM((CHUNK,), jnp.int32),
        pltpu.VMEM((CHUNK,), jnp.int32),
        pltpu.VMEM((CHUNK, D), jnp.float32),
        pltpu.VMEM_SHARED((NPAD, D), jnp.float32),
        pltpu.SemaphoreType.DMA,
    ],
)
def _sc_gather_scatter(h_hbm, src_hbm, dst_hbm, out_hbm, idx_s, idx_d, rows, agg, sem):
    cid = lax.axis_index("c")
    sid = lax.axis_index("s")
    wid = sid * NC + cid

    _zero_vmem_2d(rows, CHUNK)
    def zslice(r, carry):
        pltpu.sync_copy(rows, agg.at[pl.ds(sid * ROWS_PER_TILE + r * CHUNK, CHUNK)])
        return carry
    lax.fori_loop(0, ROWS_PER_TILE // CHUNK, zslice, 0)
    plsc.subcore_barrier()

    def body(j, carry):
        t = wid * NCHUNK + j
        pltpu.sync_copy(src_hbm.at[t], idx_s)
        pltpu.sync_copy(dst_hbm.at[t], idx_d)
        pltpu.async_copy(h_hbm.at[idx_s], rows, sem).wait()
        pltpu.sync_copy(rows, agg.at[idx_d], add=True)
        return carry
    lax.fori_loop(0, NCHUNK, body, 0)

    plsc.subcore_barrier()
    pltpu.sync_copy(
        agg.at[pl.ds(sid * ROWS_PER_TILE, ROWS_PER_TILE)],
        out_hbm.at[pl.ds(cid * NPAD + sid * ROWS_PER_TILE, ROWS_PER_TILE)],
    )


def _norm(d0, d1):
    return lax.rsqrt(jnp.maximum(d0 + d1, 1.0))


def _tc1_body(od0_ref, od1_ref, feats_ref, w1_ref, h1_ref):
    ns = _norm(od0_ref[...], od1_ref[...])
    h1_ref[...] = jnp.dot(feats_ref[...] * ns, w1_ref[...],
                          preferred_element_type=jnp.float32)


def _tc2_body(a0_ref, a1_ref, id0_ref, id1_ref, od0_ref, od1_ref,
              b1_ref, w2_ref, h2_ref):
    nd = _norm(id0_ref[...], id1_ref[...])
    ns = _norm(od0_ref[...], od1_ref[...])
    x = jax.nn.relu((a0_ref[...] + a1_ref[...]) * nd + b1_ref[...][None, :])
    h2_ref[...] = jnp.dot(x * ns, w2_ref[...], preferred_element_type=jnp.float32)


def _tc3_body(a0_ref, a1_ref, id0_ref, id1_ref, b2_ref, out_ref):
    nd = _norm(id0_ref[...], id1_ref[...])
    out_ref[...] = (a0_ref[...] + a1_ref[...]) * nd + b2_ref[...][None, :]


_tc1 = pl.pallas_call(
    _tc1_body, out_shape=jax.ShapeDtypeStruct((NPAD, D), jnp.float32))
_tc2 = pl.pallas_call(
    _tc2_body, out_shape=jax.ShapeDtypeStruct((NPAD, D), jnp.float32))
_tc3 = pl.pallas_call(
    _tc3_body, out_shape=jax.ShapeDtypeStruct((NPAD, D), jnp.float32))


def kernel(feats, edge_index, W1, b1, W2, b2):
    src = edge_index[0].astype(jnp.int32)
    dst = edge_index[1].astype(jnp.int32)

    pad_e = EPAD - N_EDGES
    pad_idx = jnp.full((pad_e,), NPAD - 1, jnp.int32)
    src_p = jnp.concatenate([src, pad_idx])
    dst_p = jnp.concatenate([dst, pad_idx])
    src_tab = src_p.reshape(NW * NCHUNK, CHUNK)
    dst_tab = dst_p.reshape(NW * NCHUNK, CHUNK)
    deg_idx = jnp.concatenate([src_p, dst_p + NPAD]).reshape(
        NW * DEG_CHUNKS, CHUNK)

    feats_pad = jnp.zeros((NPAD, D), jnp.float32).at[:N_NODES].set(feats)

    deg_parts = _sc_degrees(deg_idx)
    od0 = deg_parts[0, :NPAD].reshape(NPAD, 1)
    od1 = deg_parts[1, :NPAD].reshape(NPAD, 1)
    id0 = deg_parts[0, NPAD:].reshape(NPAD, 1)
    id1 = deg_parts[1, NPAD:].reshape(NPAD, 1)

    h1 = _tc1(od0, od1, feats_pad, W1)
    agg1 = _sc_gather_scatter(h1, src_tab, dst_tab)
    h2 = _tc2(agg1[:NPAD], agg1[NPAD:], id0, id1, od0, od1, b1, W2)
    agg2 = _sc_gather_scatter(h2, src_tab, dst_tab)
    out = _tc3(agg2[:NPAD], agg2[NPAD:], id0, id1, b2)
    return out[:N_NODES]


# speedup vs baseline: 6.2258x; 2.0988x over previous
import functools

import jax
import jax.numpy as jnp
from jax import lax
from jax.experimental import pallas as pl
from jax.experimental.pallas import tpu as pltpu
from jax.experimental.pallas import tpu_sc as plsc

N_NODES = 10000
D = 128
N_EDGES = 320000

NC = 2
NS = 16
NW = NC * NS

CH = 128
NCH = 79
NCHTOT = NW * NCH
EPAD = NCHTOT * CH
DISCARD = N_NODES
AGG_N = 10016
RPT = 624
TAIL = AGG_N - NS * RPT

DEGCH = 160
DEG_ROWS = NW * DEGCH
DEG_DST_OFF = 10240
DEGN = 2 * DEG_DST_OFF

_MESH = plsc.VectorSubcoreMesh(core_axis_name="c", subcore_axis_name="s")


def _zero_vmem_2d(ref, n_rows):
    def body(i, carry):
        ref[i // 8, pl.ds((i % 8) * 16, 16)] = jnp.zeros((16,), jnp.float32)
        return carry
    lax.fori_loop(0, n_rows * 8, body, 0)


def _zero_vmem_1d(ref, n):
    def body(i, carry):
        ref[pl.ds(i * 16, 16)] = jnp.zeros((16,), jnp.float32)
        return carry
    lax.fori_loop(0, n // 16, body, 0)


DEG_FIRE = 8
DEG_TAIL = DEGCH % DEG_FIRE

@functools.partial(
    pl.kernel,
    out_type=jax.ShapeDtypeStruct((NC, DEGN), jnp.float32),
    mesh=_MESH,
    scratch_types=[
        pltpu.VMEM((DEGCH, CH), jnp.int32),
        pltpu.VMEM((CH,), jnp.float32),
        pltpu.VMEM_SHARED((DEGN,), jnp.float32),
        pltpu.SemaphoreType.DMA,
    ],
)
def _sc_degrees(deg_idx_hbm, out_hbm, idx_all, ones_v, deg_sh, sem):
    cid = lax.axis_index("c")
    sid = lax.axis_index("s")
    wid = sid * NC + cid

    pltpu.sync_copy(deg_idx_hbm.at[pl.ds(wid * DEGCH, DEGCH)], idx_all)

    per_tile = DEGN // NS
    _zero_vmem_1d(ones_v, CH)
    def zslice(r, carry):
        pltpu.sync_copy(ones_v, deg_sh.at[pl.ds(sid * per_tile + r * CH, CH)])
        return carry
    lax.fori_loop(0, per_tile // CH, zslice, 0)
    def ones_body(i, carry):
        ones_v[pl.ds(i * 16, 16)] = jnp.ones((16,), jnp.float32)
        return carry
    lax.fori_loop(0, CH // 16, ones_body, 0)

    plsc.subcore_barrier()

    def body(g, carry):
        for b in range(DEG_FIRE):
            pltpu.async_copy(ones_v, deg_sh.at[idx_all.at[g * DEG_FIRE + b]],
                             sem, add=True)
        for b in range(DEG_FIRE):
            pltpu.make_async_copy(deg_idx_hbm.at[0], ones_v, sem).wait()
        return carry
    lax.fori_loop(0, DEGCH // DEG_FIRE, body, 0)
    for b in range(DEG_TAIL):
        pltpu.async_copy(ones_v, deg_sh.at[idx_all.at[DEGCH - DEG_TAIL + b]],
                         sem, add=True)
    for b in range(DEG_TAIL):
        pltpu.make_async_copy(deg_idx_hbm.at[0], ones_v, sem).wait()

    plsc.subcore_barrier()
    pltpu.sync_copy(
        deg_sh.at[pl.ds(sid * per_tile, per_tile)],
        out_hbm.at[cid, pl.ds(sid * per_tile, per_tile)],
    )


@functools.partial(
    pl.kernel,
    out_type=jax.ShapeDtypeStruct((NC * AGG_N, D), jnp.float32),
    mesh=_MESH,
    scratch_types=[
        pltpu.VMEM((CH, D), jnp.float32),
        pltpu.VMEM((CH, D), jnp.float32),
        pltpu.VMEM((CH, D), jnp.float32),
        pltpu.VMEM((2, 2, CH), jnp.int32),
        pltpu.VMEM((2, 2, CH), jnp.int32),
        pltpu.VMEM((2, 2, CH), jnp.int32),
        pltpu.VMEM_SHARED((AGG_N, D), jnp.float32),
        pltpu.SemaphoreType.DMA,
        pltpu.SemaphoreType.DMA,
        pltpu.SemaphoreType.DMA,
        pltpu.SemaphoreType.DMA,
    ],
)
def _sc_gather_scatter(h_hbm, tab_hbm, out_hbm,
                       buf0, buf1, buf2, ip0, ip1, ip2, agg,
                       sg0, sg1, sg2, ss):
    cid = lax.axis_index("c")
    sid = lax.axis_index("s")
    wid = sid * NC + cid
    base = wid * NCH
    bufs = (buf0, buf1, buf2)
    ips = (ip0, ip1, ip2)
    sgs = (sg0, sg1, sg2)

    def wait_g(b):
        pltpu.make_async_copy(h_hbm.at[pl.ds(0, CH)], bufs[b], sgs[b]).wait()

    def wait_s(b):
        pltpu.make_async_copy(h_hbm.at[pl.ds(0, CH)], bufs[b], ss).wait()

    _zero_vmem_2d(buf0, CH)
    for r in range(4):
        pltpu.sync_copy(buf0, agg.at[pl.ds(sid * RPT + r * CH, CH)])
    pltpu.sync_copy(buf0.at[pl.ds(0, RPT - 4 * CH)],
                    agg.at[pl.ds(sid * RPT + 4 * CH, RPT - 4 * CH)])
    @pl.when(sid == 0)
    def _zero_tail():
        pltpu.sync_copy(buf0.at[pl.ds(0, TAIL)],
                        agg.at[pl.ds(NS * RPT, TAIL)])

    for b in range(3):
        pltpu.sync_copy(tab_hbm.at[base + b], ips[b].at[0])
        pltpu.async_copy(h_hbm.at[ips[b].at[0, 0]], bufs[b], sgs[b])

    plsc.subcore_barrier()

    def outer(g2, carry):
        for u in range(6):
            b = u % 3
            q = (u // 3) % 2
            j = g2 * 6 + u
            wait_g(b)
            pltpu.async_copy(bufs[b], agg.at[ips[b].at[q, 1]], ss, add=True)
            pltpu.sync_copy(tab_hbm.at[base + j + 3], ips[b].at[1 - q])
            wait_s(b)
            pltpu.async_copy(h_hbm.at[ips[b].at[1 - q, 0]], bufs[b], sgs[b])
        return carry
    lax.fori_loop(0, 12, outer, 0)

    for u in range(3):
        b = u
        j = 72 + u
        wait_g(b)
        pltpu.async_copy(bufs[b], agg.at[ips[b].at[0, 1]], ss, add=True)
        pltpu.sync_copy(tab_hbm.at[base + j + 3], ips[b].at[1])
        wait_s(b)
        pltpu.async_copy(h_hbm.at[ips[b].at[1, 0]], bufs[b], sgs[b])
    wait_g(0)
    pltpu.async_copy(bufs[0], agg.at[ips[0].at[1, 1]], ss, add=True)
    pltpu.sync_copy(tab_hbm.at[base + 78], ips[0].at[0])
    wait_s(0)
    pltpu.async_copy(h_hbm.at[ips[0].at[0, 0]], bufs[0], sgs[0])
    wait_g(1)
    pltpu.sync_copy(bufs[1], agg.at[ips[1].at[1, 1]], add=True)
    wait_g(2)
    pltpu.sync_copy(bufs[2], agg.at[ips[2].at[1, 1]], add=True)
    wait_g(0)
    pltpu.sync_copy(bufs[0], agg.at[ips[0].at[0, 1]], add=True)

    plsc.subcore_barrier()
    pltpu.sync_copy(
        agg.at[pl.ds(sid * RPT, RPT)],
        out_hbm.at[pl.ds(cid * AGG_N + sid * RPT, RPT)],
    )
    @pl.when(sid == 0)
    def _write_tail():
        pltpu.sync_copy(agg.at[pl.ds(NS * RPT, TAIL)],
                        out_hbm.at[pl.ds(cid * AGG_N + NS * RPT, TAIL)])


def _norm(d0, d1):
    return lax.rsqrt(jnp.maximum(d0 + d1, 1.0))


def _tc1_body(od0_ref, od1_ref, feats_ref, w1_ref, h1_ref):
    ns = _norm(od0_ref[...], od1_ref[...])
    h1_ref[...] = jnp.dot(feats_ref[...] * ns, w1_ref[...],
                          preferred_element_type=jnp.float32)


def _tc2_body(a0_ref, a1_ref, id0_ref, id1_ref, od0_ref, od1_ref,
              b1_ref, w2_ref, h2_ref):
    nd = _norm(id0_ref[...], id1_ref[...])
    ns = _norm(od0_ref[...], od1_ref[...])
    x = jax.nn.relu((a0_ref[...] + a1_ref[...]) * nd + b1_ref[...][None, :])
    h2_ref[...] = jnp.dot(x * ns, w2_ref[...], preferred_element_type=jnp.float32)


def _tc3_body(a0_ref, a1_ref, id0_ref, id1_ref, b2_ref, out_ref):
    nd = _norm(id0_ref[...], id1_ref[...])
    out_ref[...] = (a0_ref[...] + a1_ref[...]) * nd + b2_ref[...][None, :]


_tc1 = pl.pallas_call(
    _tc1_body, out_shape=jax.ShapeDtypeStruct((N_NODES, D), jnp.float32))
_tc2 = pl.pallas_call(
    _tc2_body, out_shape=jax.ShapeDtypeStruct((N_NODES, D), jnp.float32))
_tc3 = pl.pallas_call(
    _tc3_body, out_shape=jax.ShapeDtypeStruct((N_NODES, D), jnp.float32))


def kernel(feats, edge_index, W1, b1, W2, b2):
    src = edge_index[0].astype(jnp.int32)
    dst = edge_index[1].astype(jnp.int32)

    pad_e = EPAD - N_EDGES
    src_p = jnp.concatenate([src, jnp.zeros((pad_e,), jnp.int32)])
    dst_p = jnp.concatenate([dst, jnp.full((pad_e,), DISCARD, jnp.int32)])
    tab = jnp.stack([src_p.reshape(NCHTOT, CH), dst_p.reshape(NCHTOT, CH)],
                    axis=1)

    n_real_rows = 2 * (N_EDGES // CH)
    deg_idx = jnp.concatenate([
        src.reshape(-1, CH),
        (dst + DEG_DST_OFF).reshape(-1, CH),
        jnp.full((DEG_ROWS - n_real_rows, CH), DISCARD, jnp.int32),
    ])

    deg_parts = _sc_degrees(deg_idx)
    od0 = deg_parts[0, :N_NODES].reshape(N_NODES, 1)
    od1 = deg_parts[1, :N_NODES].reshape(N_NODES, 1)
    id0 = deg_parts[0, DEG_DST_OFF:DEG_DST_OFF + N_NODES].reshape(N_NODES, 1)
    id1 = deg_parts[1, DEG_DST_OFF:DEG_DST_OFF + N_NODES].reshape(N_NODES, 1)

    h1 = _tc1(od0, od1, feats, W1)
    agg1 = _sc_gather_scatter(h1, tab)
    h2 = _tc2(agg1[:N_NODES], agg1[AGG_N:AGG_N + N_NODES],
              id0, id1, od0, od1, b1, W2)
    agg2 = _sc_gather_scatter(h2, tab)
    out = _tc3(agg2[:N_NODES], agg2[AGG_N:AGG_N + N_NODES], id0, id1, b2)
    return out


# speedup vs baseline: 6.3583x; 1.0213x over previous
import functools

import jax
import jax.numpy as jnp
from jax import lax
from jax.experimental import pallas as pl
from jax.experimental.pallas import tpu as pltpu
from jax.experimental.pallas import tpu_sc as plsc

N_NODES = 10000
D = 128
N_EDGES = 320000

NC = 2
NS = 16
NW = NC * NS

CH = 128
NCH = 79
NCHTOT = NW * NCH
EPAD = NCHTOT * CH
DISCARD = N_NODES
AGG_N = 10016
RPT = 624
TAIL = AGG_N - NS * RPT

DEGCH = 160
DEG_ROWS = NW * DEGCH
DEG_DST_OFF = 10240
DEGN = 2 * DEG_DST_OFF

_MESH = plsc.VectorSubcoreMesh(core_axis_name="c", subcore_axis_name="s")


def _zero_vmem_2d(ref, n_rows):
    def body(i, carry):
        ref[i // 8, pl.ds((i % 8) * 16, 16)] = jnp.zeros((16,), jnp.float32)
        return carry
    lax.fori_loop(0, n_rows * 8, body, 0)


def _zero_vmem_1d(ref, n):
    def body(i, carry):
        ref[pl.ds(i * 16, 16)] = jnp.zeros((16,), jnp.float32)
        return carry
    lax.fori_loop(0, n // 16, body, 0)


DEG_FIRE = 8
DEG_TAIL = DEGCH % DEG_FIRE

@functools.partial(
    pl.kernel,
    out_type=jax.ShapeDtypeStruct((NC, DEGN), jnp.float32),
    mesh=_MESH,
    scratch_types=[
        pltpu.VMEM((DEGCH, CH), jnp.int32),
        pltpu.VMEM((CH,), jnp.float32),
        pltpu.VMEM_SHARED((DEGN,), jnp.float32),
        pltpu.SemaphoreType.DMA,
    ],
)
def _sc_degrees(deg_idx_hbm, out_hbm, idx_all, ones_v, deg_sh, sem):
    cid = lax.axis_index("c")
    sid = lax.axis_index("s")
    wid = sid * NC + cid

    pltpu.sync_copy(deg_idx_hbm.at[pl.ds(wid * DEGCH, DEGCH)], idx_all)

    per_tile = DEGN // NS
    _zero_vmem_1d(ones_v, CH)
    def zslice(r, carry):
        pltpu.sync_copy(ones_v, deg_sh.at[pl.ds(sid * per_tile + r * CH, CH)])
        return carry
    lax.fori_loop(0, per_tile // CH, zslice, 0)
    def ones_body(i, carry):
        ones_v[pl.ds(i * 16, 16)] = jnp.ones((16,), jnp.float32)
        return carry
    lax.fori_loop(0, CH // 16, ones_body, 0)

    plsc.subcore_barrier()

    def body(g, carry):
        for b in range(DEG_FIRE):
            pltpu.async_copy(ones_v, deg_sh.at[idx_all.at[g * DEG_FIRE + b]],
                             sem, add=True)
        for b in range(DEG_FIRE):
            pltpu.make_async_copy(deg_idx_hbm.at[0], ones_v, sem).wait()
        return carry
    lax.fori_loop(0, DEGCH // DEG_FIRE, body, 0)
    for b in range(DEG_TAIL):
        pltpu.async_copy(ones_v, deg_sh.at[idx_all.at[DEGCH - DEG_TAIL + b]],
                         sem, add=True)
    for b in range(DEG_TAIL):
        pltpu.make_async_copy(deg_idx_hbm.at[0], ones_v, sem).wait()

    plsc.subcore_barrier()
    pltpu.sync_copy(
        deg_sh.at[pl.ds(sid * per_tile, per_tile)],
        out_hbm.at[cid, pl.ds(sid * per_tile, per_tile)],
    )


@functools.partial(
    pl.kernel,
    out_type=jax.ShapeDtypeStruct((NC * AGG_N, D), jnp.float32),
    mesh=_MESH,
    scratch_types=[
        pltpu.VMEM((CH, D), jnp.float32),
        pltpu.VMEM((CH, D), jnp.float32),
        pltpu.VMEM((CH, D), jnp.float32),
        pltpu.VMEM((2, 2, CH), jnp.int32),
        pltpu.VMEM((2, 2, CH), jnp.int32),
        pltpu.VMEM((2, 2, CH), jnp.int32),
        pltpu.VMEM_SHARED((AGG_N, D), jnp.float32),
        pltpu.SemaphoreType.DMA,
        pltpu.SemaphoreType.DMA,
        pltpu.SemaphoreType.DMA,
        pltpu.SemaphoreType.DMA,
    ],
)
def _sc_gather_scatter(h_hbm, tab_hbm, out_hbm,
                       buf0, buf1, buf2, ip0, ip1, ip2, agg,
                       sg0, sg1, sg2, ss):
    cid = lax.axis_index("c")
    sid = lax.axis_index("s")
    wid = sid * NC + cid
    base = wid * NCH
    bufs = (buf0, buf1, buf2)
    ips = (ip0, ip1, ip2)
    sgs = (sg0, sg1, sg2)

    def wait_g(b):
        pltpu.make_async_copy(h_hbm.at[pl.ds(0, CH)], bufs[b], sgs[b]).wait()

    def wait_s(b):
        pltpu.make_async_copy(h_hbm.at[pl.ds(0, CH)], bufs[b], ss).wait()

    _zero_vmem_2d(buf0, CH)
    for r in range(4):
        pltpu.sync_copy(buf0, agg.at[pl.ds(sid * RPT + r * CH, CH)])
    pltpu.sync_copy(buf0.at[pl.ds(0, RPT - 4 * CH)],
                    agg.at[pl.ds(sid * RPT + 4 * CH, RPT - 4 * CH)])
    @pl.when(sid == 0)
    def _zero_tail():
        pltpu.sync_copy(buf0.at[pl.ds(0, TAIL)],
                        agg.at[pl.ds(NS * RPT, TAIL)])

    for b in range(3):
        pltpu.sync_copy(tab_hbm.at[base + b], ips[b].at[0])
        pltpu.async_copy(h_hbm.at[ips[b].at[0, 0]], bufs[b], sgs[b])

    plsc.subcore_barrier()

    def outer(g2, carry):
        for u in range(6):
            b = u % 3
            q = (u // 3) % 2
            j = g2 * 6 + u
            wait_g(b)
            pltpu.async_copy(bufs[b], agg.at[ips[b].at[q, 1]], ss, add=True)
            pltpu.sync_copy(tab_hbm.at[base + j + 3], ips[b].at[1 - q])
            wait_s(b)
            pltpu.async_copy(h_hbm.at[ips[b].at[1 - q, 0]], bufs[b], sgs[b])
        return carry
    lax.fori_loop(0, 12, outer, 0)

    for u in range(3):
        b = u
        j = 72 + u
        wait_g(b)
        pltpu.async_copy(bufs[b], agg.at[ips[b].at[0, 1]], ss, add=True)
        pltpu.sync_copy(tab_hbm.at[base + j + 3], ips[b].at[1])
        wait_s(b)
        pltpu.async_copy(h_hbm.at[ips[b].at[1, 0]], bufs[b], sgs[b])
    wait_g(0)
    pltpu.async_copy(bufs[0], agg.at[ips[0].at[1, 1]], ss, add=True)
    pltpu.sync_copy(tab_hbm.at[base + 78], ips[0].at[0])
    wait_s(0)
    pltpu.async_copy(h_hbm.at[ips[0].at[0, 0]], bufs[0], sgs[0])
    wait_g(1)
    pltpu.sync_copy(bufs[1], agg.at[ips[1].at[1, 1]], add=True)
    wait_g(2)
    pltpu.sync_copy(bufs[2], agg.at[ips[2].at[1, 1]], add=True)
    wait_g(0)
    pltpu.sync_copy(bufs[0], agg.at[ips[0].at[0, 1]], add=True)

    plsc.subcore_barrier()
    pltpu.sync_copy(
        agg.at[pl.ds(sid * RPT, RPT)],
        out_hbm.at[pl.ds(cid * AGG_N + sid * RPT, RPT)],
    )
    @pl.when(sid == 0)
    def _write_tail():
        pltpu.sync_copy(agg.at[pl.ds(NS * RPT, TAIL)],
                        out_hbm.at[pl.ds(cid * AGG_N + NS * RPT, TAIL)])


def _norm(d0, d1):
    return lax.rsqrt(jnp.maximum(d0 + d1, 1.0))


def _tc1_body(od0_ref, od1_ref, feats_ref, w1_ref, h1_ref):
    ns = _norm(od0_ref[...], od1_ref[...])
    h1_ref[...] = jnp.dot(feats_ref[...] * ns, w1_ref[...],
                          preferred_element_type=jnp.float32)


def _tc2_body(a0_ref, a1_ref, id0_ref, id1_ref, od0_ref, od1_ref,
              b1_ref, w2_ref, h2_ref):
    nd = _norm(id0_ref[...], id1_ref[...])
    ns = _norm(od0_ref[...], od1_ref[...])
    x = jax.nn.relu((a0_ref[...] + a1_ref[...]) * nd + b1_ref[...][None, :])
    h2_ref[...] = jnp.dot(x * ns, w2_ref[...], preferred_element_type=jnp.float32)


def _tc3_body(a0_ref, a1_ref, id0_ref, id1_ref, b2_ref, out_ref):
    nd = _norm(id0_ref[...], id1_ref[...])
    out_ref[...] = (a0_ref[...] + a1_ref[...]) * nd + b2_ref[...][None, :]


_tc1 = pl.pallas_call(
    _tc1_body, out_shape=jax.ShapeDtypeStruct((N_NODES, D), jnp.float32))
_tc2 = pl.pallas_call(
    _tc2_body, out_shape=jax.ShapeDtypeStruct((N_NODES, D), jnp.float32))
_tc3 = pl.pallas_call(
    _tc3_body, out_shape=jax.ShapeDtypeStruct((N_NODES, D), jnp.float32))


def kernel(feats, edge_index, W1, b1, W2, b2):
    src = edge_index[0].astype(jnp.int32)
    dst = edge_index[1].astype(jnp.int32)

    pad_e = EPAD - N_EDGES
    src_p = jnp.concatenate([src, jnp.zeros((pad_e,), jnp.int32)])
    dst_p = jnp.concatenate(
        [dst, DISCARD + (jnp.arange(pad_e, dtype=jnp.int32) % (AGG_N - N_NODES))])
    tab = jnp.stack([src_p.reshape(NCHTOT, CH), dst_p.reshape(NCHTOT, CH)],
                    axis=1)

    n_real_rows = 2 * (N_EDGES // CH)
    n_pad_rows = DEG_ROWS - n_real_rows
    deg_pad = N_NODES + (jnp.arange(n_pad_rows * CH, dtype=jnp.int32) % CH)
    deg_idx = jnp.concatenate([
        src.reshape(-1, CH),
        (dst + DEG_DST_OFF).reshape(-1, CH),
        deg_pad.reshape(n_pad_rows, CH),
    ])

    deg_parts = _sc_degrees(deg_idx)
    od0 = deg_parts[0, :N_NODES].reshape(N_NODES, 1)
    od1 = deg_parts[1, :N_NODES].reshape(N_NODES, 1)
    id0 = deg_parts[0, DEG_DST_OFF:DEG_DST_OFF + N_NODES].reshape(N_NODES, 1)
    id1 = deg_parts[1, DEG_DST_OFF:DEG_DST_OFF + N_NODES].reshape(N_NODES, 1)

    h1 = _tc1(od0, od1, feats, W1)
    agg1 = _sc_gather_scatter(h1, tab)
    h2 = _tc2(agg1[:N_NODES], agg1[AGG_N:AGG_N + N_NODES],
              id0, id1, od0, od1, b1, W2)
    agg2 = _sc_gather_scatter(h2, tab)
    out = _tc3(agg2[:N_NODES], agg2[AGG_N:AGG_N + N_NODES], id0, id1, b2)
    return out


# speedup vs baseline: 6.7310x; 1.0586x over previous
import functools

import jax
import jax.numpy as jnp
from jax import lax
from jax.experimental import pallas as pl
from jax.experimental.pallas import tpu as pltpu
from jax.experimental.pallas import tpu_sc as plsc

N_NODES = 10000
D = 128
N_EDGES = 320000

NC = 2
NS = 16
NW = NC * NS

CH = 128
NCHTOT = 2528
EPAD = NCHTOT * CH
FAST_CID = 0
NCH_FAST = 115
NCH_SLOW = 43
DISCARD = N_NODES
AGG_N = 10016
RPT = 624
TAIL = AGG_N - NS * RPT

DEGCH = 160
DEG_ROWS = NW * DEGCH
DEG_DST_OFF = 10240
DEGN = 2 * DEG_DST_OFF

_MESH = plsc.VectorSubcoreMesh(core_axis_name="c", subcore_axis_name="s")


def _zero_vmem_2d(ref, n_rows):
    def body(i, carry):
        ref[i // 8, pl.ds((i % 8) * 16, 16)] = jnp.zeros((16,), jnp.float32)
        return carry
    lax.fori_loop(0, n_rows * 8, body, 0)


def _zero_vmem_1d(ref, n):
    def body(i, carry):
        ref[pl.ds(i * 16, 16)] = jnp.zeros((16,), jnp.float32)
        return carry
    lax.fori_loop(0, n // 16, body, 0)


DEG_FIRE = 8
DEG_TAIL = DEGCH % DEG_FIRE

@functools.partial(
    pl.kernel,
    out_type=jax.ShapeDtypeStruct((NC, DEGN), jnp.float32),
    mesh=_MESH,
    scratch_types=[
        pltpu.VMEM((DEGCH, CH), jnp.int32),
        pltpu.VMEM((CH,), jnp.float32),
        pltpu.VMEM_SHARED((DEGN,), jnp.float32),
        pltpu.SemaphoreType.DMA,
    ],
)
def _sc_degrees(deg_idx_hbm, out_hbm, idx_all, ones_v, deg_sh, sem):
    cid = lax.axis_index("c")
    sid = lax.axis_index("s")
    wid = sid * NC + cid

    pltpu.sync_copy(deg_idx_hbm.at[pl.ds(wid * DEGCH, DEGCH)], idx_all)

    per_tile = DEGN // NS
    _zero_vmem_1d(ones_v, CH)
    def zslice(r, carry):
        pltpu.sync_copy(ones_v, deg_sh.at[pl.ds(sid * per_tile + r * CH, CH)])
        return carry
    lax.fori_loop(0, per_tile // CH, zslice, 0)
    def ones_body(i, carry):
        ones_v[pl.ds(i * 16, 16)] = jnp.ones((16,), jnp.float32)
        return carry
    lax.fori_loop(0, CH // 16, ones_body, 0)

    plsc.subcore_barrier()

    def body(g, carry):
        for b in range(DEG_FIRE):
            pltpu.async_copy(ones_v, deg_sh.at[idx_all.at[g * DEG_FIRE + b]],
                             sem, add=True)
        for b in range(DEG_FIRE):
            pltpu.make_async_copy(deg_idx_hbm.at[0], ones_v, sem).wait()
        return carry
    lax.fori_loop(0, DEGCH // DEG_FIRE, body, 0)
    for b in range(DEG_TAIL):
        pltpu.async_copy(ones_v, deg_sh.at[idx_all.at[DEGCH - DEG_TAIL + b]],
                         sem, add=True)
    for b in range(DEG_TAIL):
        pltpu.make_async_copy(deg_idx_hbm.at[0], ones_v, sem).wait()

    plsc.subcore_barrier()
    pltpu.sync_copy(
        deg_sh.at[pl.ds(sid * per_tile, per_tile)],
        out_hbm.at[cid, pl.ds(sid * per_tile, per_tile)],
    )


@functools.partial(
    pl.kernel,
    out_type=jax.ShapeDtypeStruct((NC * AGG_N, D), jnp.float32),
    mesh=_MESH,
    scratch_types=[
        pltpu.VMEM((CH, D), jnp.float32),
        pltpu.VMEM((CH, D), jnp.float32),
        pltpu.VMEM((CH, D), jnp.float32),
        pltpu.VMEM((2, 2, CH), jnp.int32),
        pltpu.VMEM((2, 2, CH), jnp.int32),
        pltpu.VMEM((2, 2, CH), jnp.int32),
        pltpu.VMEM_SHARED((AGG_N, D), jnp.float32),
        pltpu.SemaphoreType.DMA,
        pltpu.SemaphoreType.DMA,
        pltpu.SemaphoreType.DMA,
        pltpu.SemaphoreType.DMA,
    ],
)
def _sc_gather_scatter(h_hbm, tab_hbm, out_hbm,
                       buf0, buf1, buf2, ip0, ip1, ip2, agg,
                       sg0, sg1, sg2, ss):
    cid = lax.axis_index("c")
    sid = lax.axis_index("s")
    is_fast = cid == FAST_CID
    base = jnp.where(is_fast, sid * NCH_FAST,
                     NS * NCH_FAST + sid * NCH_SLOW)
    n_groups = jnp.where(is_fast, (NCH_FAST - 7) // 6, (NCH_SLOW - 7) // 6)
    m = base + 6 * n_groups
    bufs = (buf0, buf1, buf2)
    ips = (ip0, ip1, ip2)
    sgs = (sg0, sg1, sg2)

    def wait_g(b):
        pltpu.make_async_copy(h_hbm.at[pl.ds(0, CH)], bufs[b], sgs[b]).wait()

    def wait_s(b):
        pltpu.make_async_copy(h_hbm.at[pl.ds(0, CH)], bufs[b], ss).wait()

    _zero_vmem_2d(buf0, CH)
    for r in range(4):
        pltpu.sync_copy(buf0, agg.at[pl.ds(sid * RPT + r * CH, CH)])
    pltpu.sync_copy(buf0.at[pl.ds(0, RPT - 4 * CH)],
                    agg.at[pl.ds(sid * RPT + 4 * CH, RPT - 4 * CH)])
    @pl.when(sid == 0)
    def _zero_tail():
        pltpu.sync_copy(buf0.at[pl.ds(0, TAIL)],
                        agg.at[pl.ds(NS * RPT, TAIL)])

    for b in range(3):
        pltpu.sync_copy(tab_hbm.at[base + b], ips[b].at[0])
        pltpu.async_copy(h_hbm.at[ips[b].at[0, 0]], bufs[b], sgs[b])

    plsc.subcore_barrier()

    def outer(g2, carry):
        for u in range(6):
            b = u % 3
            q = (u // 3) % 2
            j = g2 * 6 + u
            wait_g(b)
            pltpu.async_copy(bufs[b], agg.at[ips[b].at[q, 1]], ss, add=True)
            pltpu.sync_copy(tab_hbm.at[base + j + 3], ips[b].at[1 - q])
            wait_s(b)
            pltpu.async_copy(h_hbm.at[ips[b].at[1 - q, 0]], bufs[b], sgs[b])
        return carry
    lax.fori_loop(0, n_groups, outer, 0)

    for u in range(3):
        b = u
        wait_g(b)
        pltpu.async_copy(bufs[b], agg.at[ips[b].at[0, 1]], ss, add=True)
        pltpu.sync_copy(tab_hbm.at[m + u + 3], ips[b].at[1])
        wait_s(b)
        pltpu.async_copy(h_hbm.at[ips[b].at[1, 0]], bufs[b], sgs[b])
    wait_g(0)
    pltpu.async_copy(bufs[0], agg.at[ips[0].at[1, 1]], ss, add=True)
    pltpu.sync_copy(tab_hbm.at[m + 6], ips[0].at[0])
    wait_s(0)
    pltpu.async_copy(h_hbm.at[ips[0].at[0, 0]], bufs[0], sgs[0])
    wait_g(1)
    pltpu.sync_copy(bufs[1], agg.at[ips[1].at[1, 1]], add=True)
    wait_g(2)
    pltpu.sync_copy(bufs[2], agg.at[ips[2].at[1, 1]], add=True)
    wait_g(0)
    pltpu.sync_copy(bufs[0], agg.at[ips[0].at[0, 1]], add=True)

    plsc.subcore_barrier()
    pltpu.sync_copy(
        agg.at[pl.ds(sid * RPT, RPT)],
        out_hbm.at[pl.ds(cid * AGG_N + sid * RPT, RPT)],
    )
    @pl.when(sid == 0)
    def _write_tail():
        pltpu.sync_copy(agg.at[pl.ds(NS * RPT, TAIL)],
                        out_hbm.at[pl.ds(cid * AGG_N + NS * RPT, TAIL)])


def _norm(d0, d1):
    return lax.rsqrt(jnp.maximum(d0 + d1, 1.0))


def _tc1_body(od0_ref, od1_ref, feats_ref, w1_ref, h1_ref):
    ns = _norm(od0_ref[...], od1_ref[...])
    h1_ref[...] = jnp.dot(feats_ref[...] * ns, w1_ref[...],
                          preferred_element_type=jnp.float32)


def _tc2_body(a0_ref, a1_ref, id0_ref, id1_ref, od0_ref, od1_ref,
              b1_ref, w2_ref, h2_ref):
    nd = _norm(id0_ref[...], id1_ref[...])
    ns = _norm(od0_ref[...], od1_ref[...])
    x = jax.nn.relu((a0_ref[...] + a1_ref[...]) * nd + b1_ref[...][None, :])
    h2_ref[...] = jnp.dot(x * ns, w2_ref[...], preferred_element_type=jnp.float32)


def _tc3_body(a0_ref, a1_ref, id0_ref, id1_ref, b2_ref, out_ref):
    nd = _norm(id0_ref[...], id1_ref[...])
    out_ref[...] = (a0_ref[...] + a1_ref[...]) * nd + b2_ref[...][None, :]


_tc1 = pl.pallas_call(
    _tc1_body, out_shape=jax.ShapeDtypeStruct((N_NODES, D), jnp.float32))
_tc2 = pl.pallas_call(
    _tc2_body, out_shape=jax.ShapeDtypeStruct((N_NODES, D), jnp.float32))
_tc3 = pl.pallas_call(
    _tc3_body, out_shape=jax.ShapeDtypeStruct((N_NODES, D), jnp.float32))


def kernel(feats, edge_index, W1, b1, W2, b2):
    src = edge_index[0].astype(jnp.int32)
    dst = edge_index[1].astype(jnp.int32)

    pad_e = EPAD - N_EDGES
    src_p = jnp.concatenate([src, jnp.zeros((pad_e,), jnp.int32)])
    dst_p = jnp.concatenate(
        [dst, DISCARD + (jnp.arange(pad_e, dtype=jnp.int32) % (AGG_N - N_NODES))])
    tab = jnp.stack([src_p.reshape(NCHTOT, CH), dst_p.reshape(NCHTOT, CH)],
                    axis=1)

    n_real_rows = 2 * (N_EDGES // CH)
    n_pad_rows = DEG_ROWS - n_real_rows
    deg_pad = N_NODES + (jnp.arange(n_pad_rows * CH, dtype=jnp.int32) % CH)
    deg_idx = jnp.concatenate([
        src.reshape(-1, CH),
        (dst + DEG_DST_OFF).reshape(-1, CH),
        deg_pad.reshape(n_pad_rows, CH),
    ])

    deg_parts = _sc_degrees(deg_idx)
    od0 = deg_parts[0, :N_NODES].reshape(N_NODES, 1)
    od1 = deg_parts[1, :N_NODES].reshape(N_NODES, 1)
    id0 = deg_parts[0, DEG_DST_OFF:DEG_DST_OFF + N_NODES].reshape(N_NODES, 1)
    id1 = deg_parts[1, DEG_DST_OFF:DEG_DST_OFF + N_NODES].reshape(N_NODES, 1)

    h1 = _tc1(od0, od1, feats, W1)
    agg1 = _sc_gather_scatter(h1, tab)
    h2 = _tc2(agg1[:N_NODES], agg1[AGG_N:AGG_N + N_NODES],
              id0, id1, od0, od1, b1, W2)
    agg2 = _sc_gather_scatter(h2, tab)
    out = _tc3(agg2[:N_NODES], agg2[AGG_N:AGG_N + N_NODES], id0, id1, b2)
    return out
